# baseline (device time: 19022 ns/iter reference)
import jax
import jax.numpy as jnp
from jax import lax
from jax.experimental import pallas as pl
from jax.experimental.pallas import tpu as pltpu


def kernel(Q, K, V):
    b, sq, h, d = Q.shape
    _, skv, _, _ = K.shape
    hd = h * d
    scale = d ** -0.5

    KT = K.transpose(0, 2, 3, 1).reshape(b, hd, skv)
    VT = V.transpose(0, 2, 3, 1).reshape(b, hd, skv)

    def copy_in(src, dst, sems, bi, slot):
        return pltpu.make_async_copy(
            src.at[bi], dst.at[slot], sems.at[slot]
        )

    def body(q_ref, k_hbm, v_hbm, out_ref, kbuf, vbuf, send_buf, recv_buf,
             ksems, vsems, send_sem, recv_sem):
        bi = pl.program_id(0)
        slot = lax.rem(bi, 2)
        nslot = 1 - slot

        eye8 = (
            lax.broadcasted_iota(jnp.int32, (h, h), 0)
            == lax.broadcasted_iota(jnp.int32, (h, h), 1)
        ).astype(jnp.float32)

        my_x = lax.axis_index("x")
        my_y = lax.axis_index("y")
        my_z = lax.axis_index("z")
        nbr = (1 - my_x, my_y, my_z)
        barrier = pltpu.get_barrier_semaphore()

        @pl.when(bi == 0)
        def _():
            pl.semaphore_signal(barrier, inc=1, device_id=nbr,
                                device_id_type=pl.DeviceIdType.MESH)
            copy_in(k_hbm, kbuf, ksems, 0, 0).start()
            copy_in(v_hbm, vbuf, vsems, 0, 0).start()

        @pl.when(bi < b - 1)
        def _():
            copy_in(k_hbm, kbuf, ksems, bi + 1, nslot).start()
            copy_in(v_hbm, vbuf, vsems, bi + 1, nslot).start()

        with jax.named_scope("wait_in"):
            copy_in(k_hbm, kbuf, ksems, bi, slot).wait()
            copy_in(v_hbm, vbuf, vsems, bi, slot).wait()

        with jax.named_scope("dot_s"):
            q = q_ref[bi, 0]
            qbt = (eye8[:, :, None] * q[:, None, :]).reshape(h, hd)
            kt = kbuf[slot]
            st = lax.dot_general(
                qbt, kt, (((1,), (0,)), ((), ())),
                preferred_element_type=jnp.float32,
            ) * scale
            p = jnp.exp(st)

        with jax.named_scope("dot_av"):
            vt = vbuf[slot]
            a = lax.dot_general(
                p, vt, (((1,), (1,)), ((), ())),
                preferred_element_type=jnp.float32,
            )
            lb = lax.dot_general(
                p, jnp.ones((skv, d), jnp.float32), (((1,), (0,)), ((), ())),
                preferred_element_type=jnp.float32,
            )

        with jax.named_scope("pack"):
            a3 = a.reshape(h, h, d)
            abh = jnp.sum(a3 * eye8[:, :, None], axis=1)
            send_buf[0, bi] = abh
            send_buf[1, bi] = lb

        @pl.when(bi == b - 1)
        def _():
            with jax.named_scope("exchange"):
                pl.semaphore_wait(barrier, 1)
                rdma = pltpu.make_async_remote_copy(
                    src_ref=send_buf,
                    dst_ref=recv_buf,
                    send_sem=send_sem,
                    recv_sem=recv_sem,
                    device_id=nbr,
                    device_id_type=pl.DeviceIdType.MESH,
                )
                rdma.start()
                rdma.wait()

            with jax.named_scope("combine"):
                acc = send_buf[0] + recv_buf[0]
                l_all = send_buf[1] + recv_buf[1]
                out_ref[...] = (acc / l_all).reshape(b, sq, h, d)

    return pl.pallas_call(
        body,
        grid=(b,),
        out_shape=jax.ShapeDtypeStruct((b, sq, h, d), jnp.float32),
        in_specs=[
            pl.BlockSpec((b, sq, h, d), lambda i: (0, 0, 0, 0),
                         memory_space=pltpu.VMEM),
            pl.BlockSpec(memory_space=pl.ANY),
            pl.BlockSpec(memory_space=pl.ANY),
        ],
        out_specs=pl.BlockSpec((b, sq, h, d), lambda i: (0, 0, 0, 0),
                               memory_space=pltpu.VMEM),
        scratch_shapes=[
            pltpu.VMEM((2, hd, skv), jnp.float32),
            pltpu.VMEM((2, hd, skv), jnp.float32),
            pltpu.VMEM((2, b, h, d), jnp.float32),
            pltpu.VMEM((2, b, h, d), jnp.float32),
            pltpu.SemaphoreType.DMA((2,)),
            pltpu.SemaphoreType.DMA((2,)),
            pltpu.SemaphoreType.DMA,
            pltpu.SemaphoreType.DMA,
        ],
        compiler_params=pltpu.CompilerParams(collective_id=0),
    )(Q, KT, VT)


# device time: 17014 ns/iter; 1.1180x vs baseline; 1.1180x over previous
import jax
import jax.numpy as jnp
from jax import lax
from jax.experimental import pallas as pl
from jax.experimental.pallas import tpu as pltpu


def kernel(Q, K, V):
    b, sq, h, d = Q.shape
    _, skv, _, _ = K.shape
    hd = h * d
    scale = d ** -0.5

    KT = K.transpose(0, 2, 3, 1).reshape(b, hd, skv)
    VT = V.transpose(0, 2, 3, 1).reshape(b, hd, skv)

    def body(q_ref, k_ref, v_ref, out_ref, send_buf, recv_buf,
             send_sem, recv_sem):
        bi = pl.program_id(0)

        eye8 = (
            lax.broadcasted_iota(jnp.int32, (h, h), 0)
            == lax.broadcasted_iota(jnp.int32, (h, h), 1)
        ).astype(jnp.float32)

        my_x = lax.axis_index("x")
        my_y = lax.axis_index("y")
        my_z = lax.axis_index("z")
        nbr = (1 - my_x, my_y, my_z)
        barrier = pltpu.get_barrier_semaphore()

        @pl.when(bi == 0)
        def _():
            pl.semaphore_signal(barrier, inc=1, device_id=nbr,
                                device_id_type=pl.DeviceIdType.MESH)

        with jax.named_scope("vpu_s"):
            q = q_ref[bi, 0]
            kt3 = k_ref[0].reshape(h, d, skv)
            t = kt3 * q[:, :, None]
            st = jnp.sum(t, axis=1) * scale
            p = jnp.exp(st)
            lb = jnp.sum(p, axis=1, keepdims=True)

        with jax.named_scope("dot_av"):
            vt = v_ref[0]
            a = lax.dot_general(
                p, vt, (((1,), (1,)), ((), ())),
                preferred_element_type=jnp.float32,
            )

        with jax.named_scope("pack"):
            a3 = a.reshape(h, h, d)
            abh = jnp.sum(a3 * eye8[:, :, None], axis=1)
            send_buf[0, bi] = abh
            send_buf[1, bi] = jnp.broadcast_to(lb, (h, d))

        @pl.when(bi == b - 1)
        def _():
            with jax.named_scope("exchange"):
                pl.semaphore_wait(barrier, 1)
                rdma = pltpu.make_async_remote_copy(
                    src_ref=send_buf,
                    dst_ref=recv_buf,
                    send_sem=send_sem,
                    recv_sem=recv_sem,
                    device_id=nbr,
                    device_id_type=pl.DeviceIdType.MESH,
                )
                rdma.start()
                rdma.wait()

            with jax.named_scope("combine"):
                acc = send_buf[0] + recv_buf[0]
                l_all = send_buf[1] + recv_buf[1]
                out_ref[...] = (acc / l_all).reshape(b, sq, h, d)

    return pl.pallas_call(
        body,
        grid=(b,),
        out_shape=jax.ShapeDtypeStruct((b, sq, h, d), jnp.float32),
        in_specs=[
            pl.BlockSpec((b, sq, h, d), lambda i: (0, 0, 0, 0),
                         memory_space=pltpu.MemorySpace.VMEM),
            pl.BlockSpec((1, hd, skv), lambda i: (i, 0, 0),
                         memory_space=pltpu.MemorySpace.VMEM),
            pl.BlockSpec((1, hd, skv), lambda i: (i, 0, 0),
                         memory_space=pltpu.MemorySpace.VMEM),
        ],
        out_specs=pl.BlockSpec((b, sq, h, d), lambda i: (0, 0, 0, 0),
                               memory_space=pltpu.MemorySpace.VMEM),
        scratch_shapes=[
            pltpu.VMEM((2, b, h, d), jnp.float32),
            pltpu.VMEM((2, b, h, d), jnp.float32),
            pltpu.SemaphoreType.DMA,
            pltpu.SemaphoreType.DMA,
        ],
        compiler_params=pltpu.CompilerParams(collective_id=0),
    )(Q, KT, VT)


# device time: 15869 ns/iter; 1.1987x vs baseline; 1.0722x over previous
import jax
import jax.numpy as jnp
from jax import lax
from jax.experimental import pallas as pl
from jax.experimental.pallas import tpu as pltpu


def kernel(Q, K, V):
    b, sq, h, d = Q.shape
    _, skv, _, _ = K.shape
    hd = h * d
    scale = d ** -0.5

    KT = K.transpose(0, 2, 3, 1).reshape(b, hd, skv)
    VT = V.transpose(0, 2, 3, 1).reshape(b, hd, skv)

    def body(q_ref, k_ref, v_ref, out_ref, send_buf, recv_buf,
             send_sem, recv_sem):
        bi = pl.program_id(0)

        my_x = lax.axis_index("x")
        my_y = lax.axis_index("y")
        my_z = lax.axis_index("z")
        nbr = (1 - my_x, my_y, my_z)
        barrier = pltpu.get_barrier_semaphore()

        @pl.when(bi == 0)
        def _():
            pl.semaphore_signal(barrier, inc=1, device_id=nbr,
                                device_id_type=pl.DeviceIdType.MESH)

        with jax.named_scope("vpu_s"):
            q = q_ref[bi, 0]
            kt3 = k_ref[0].reshape(h, d, skv)
            t = kt3 * q[:, :, None]
            st = jnp.sum(t, axis=1) * scale
            p = jnp.exp(st)
            lb = jnp.sum(p, axis=1, keepdims=True)

        with jax.named_scope("vpu_av"):
            vt3 = v_ref[0].reshape(h, d, skv)
            abh = jnp.sum(vt3 * p[:, None, :], axis=2)
            send_buf[0, bi] = abh
            send_buf[1, bi] = jnp.broadcast_to(lb, (h, d))

        @pl.when(bi == b - 1)
        def _():
            with jax.named_scope("exchange"):
                pl.semaphore_wait(barrier, 1)
                rdma = pltpu.make_async_remote_copy(
                    src_ref=send_buf,
                    dst_ref=recv_buf,
                    send_sem=send_sem,
                    recv_sem=recv_sem,
                    device_id=nbr,
                    device_id_type=pl.DeviceIdType.MESH,
                )
                rdma.start()
                rdma.wait()

            with jax.named_scope("combine"):
                acc = send_buf[0] + recv_buf[0]
                l_all = send_buf[1] + recv_buf[1]
                out_ref[...] = (acc / l_all).reshape(b, sq, h, d)

    return pl.pallas_call(
        body,
        grid=(b,),
        out_shape=jax.ShapeDtypeStruct((b, sq, h, d), jnp.float32),
        in_specs=[
            pl.BlockSpec((b, sq, h, d), lambda i: (0, 0, 0, 0),
                         memory_space=pltpu.MemorySpace.VMEM),
            pl.BlockSpec((1, hd, skv), lambda i: (i, 0, 0),
                         memory_space=pltpu.MemorySpace.VMEM),
            pl.BlockSpec((1, hd, skv), lambda i: (i, 0, 0),
                         memory_space=pltpu.MemorySpace.VMEM),
        ],
        out_specs=pl.BlockSpec((b, sq, h, d), lambda i: (0, 0, 0, 0),
                               memory_space=pltpu.MemorySpace.VMEM),
        scratch_shapes=[
            pltpu.VMEM((2, b, h, d), jnp.float32),
            pltpu.VMEM((2, b, h, d), jnp.float32),
            pltpu.SemaphoreType.DMA,
            pltpu.SemaphoreType.DMA,
        ],
        compiler_params=pltpu.CompilerParams(collective_id=0),
    )(Q, KT, VT)


# device time: 15560 ns/iter; 1.2225x vs baseline; 1.0199x over previous
import jax
import jax.numpy as jnp
from jax import lax
from jax.experimental import pallas as pl
from jax.experimental.pallas import tpu as pltpu


def kernel(Q, K, V):
    b, sq, h, d = Q.shape
    _, skv, _, _ = K.shape
    hd = h * d
    scale = d ** -0.5

    KT = K.transpose(0, 2, 3, 1).reshape(b, hd, skv)
    VT = V.transpose(0, 2, 3, 1).reshape(b, hd, skv)

    def body(q_ref, k_ref, v_ref, out_ref, send_buf, recv_buf,
             send_sems, recv_sems):
        bi = pl.program_id(0)

        my_x = lax.axis_index("x")
        my_y = lax.axis_index("y")
        my_z = lax.axis_index("z")
        nbr = (1 - my_x, my_y, my_z)
        barrier = pltpu.get_barrier_semaphore()

        @pl.when(bi == 0)
        def _():
            pl.semaphore_signal(barrier, inc=1, device_id=nbr,
                                device_id_type=pl.DeviceIdType.MESH)

        with jax.named_scope("vpu_s"):
            q = q_ref[bi, 0]
            kt3 = k_ref[0].reshape(h, d, skv)
            t = kt3 * q[:, :, None]
            st = jnp.sum(t, axis=1) * scale
            p = jnp.exp(st)
            lb = jnp.sum(p, axis=1, keepdims=True)

        with jax.named_scope("vpu_av"):
            vt3 = v_ref[0].reshape(h, d, skv)
            abh = jnp.sum(vt3 * p[:, None, :], axis=2)
            send_buf[bi, 0] = abh
            send_buf[bi, 1] = jnp.broadcast_to(lb, (h, d))

        def exchange(lo, n, si):
            return pltpu.make_async_remote_copy(
                src_ref=send_buf.at[pl.ds(lo, n)],
                dst_ref=recv_buf.at[pl.ds(lo, n)],
                send_sem=send_sems.at[si],
                recv_sem=recv_sems.at[si],
                device_id=nbr,
                device_id_type=pl.DeviceIdType.MESH,
            )

        @pl.when(bi == b - 2)
        def _():
            with jax.named_scope("exchange_head"):
                pl.semaphore_wait(barrier, 1)
                exchange(0, b - 1, 0).start()

        @pl.when(bi == b - 1)
        def _():
            with jax.named_scope("exchange_tail"):
                exchange(b - 1, 1, 1).start()
                exchange(0, b - 1, 0).wait()
                exchange(b - 1, 1, 1).wait()

            with jax.named_scope("combine"):
                acc = send_buf[:, 0] + recv_buf[:, 0]
                l_all = send_buf[:, 1] + recv_buf[:, 1]
                out_ref[...] = (acc / l_all).reshape(b, sq, h, d)

    return pl.pallas_call(
        body,
        grid=(b,),
        out_shape=jax.ShapeDtypeStruct((b, sq, h, d), jnp.float32),
        in_specs=[
            pl.BlockSpec((b, sq, h, d), lambda i: (0, 0, 0, 0),
                         memory_space=pltpu.MemorySpace.VMEM),
            pl.BlockSpec((1, hd, skv), lambda i: (i, 0, 0),
                         memory_space=pltpu.MemorySpace.VMEM),
            pl.BlockSpec((1, hd, skv), lambda i: (i, 0, 0),
                         memory_space=pltpu.MemorySpace.VMEM),
        ],
        out_specs=pl.BlockSpec((b, sq, h, d), lambda i: (0, 0, 0, 0),
                               memory_space=pltpu.MemorySpace.VMEM),
        scratch_shapes=[
            pltpu.VMEM((b, 2, h, d), jnp.float32),
            pltpu.VMEM((b, 2, h, d), jnp.float32),
            pltpu.SemaphoreType.DMA((2,)),
            pltpu.SemaphoreType.DMA((2,)),
        ],
        compiler_params=pltpu.CompilerParams(collective_id=0),
    )(Q, KT, VT)
